# Initial kernel scaffold; baseline (speedup 1.0000x reference)
import jax
import jax.numpy as jnp
from jax.experimental import pallas as pl


def kernel(current_neuron_state, mechanism_state, prev_state):
    raise NotImplementedError("write your pallas kernel here")


# speedup vs baseline: 1683.8672x; 1683.8672x over previous
import jax
import jax.numpy as jnp
from jax import lax
from jax.experimental import pallas as pl
from jax.experimental.pallas import tpu as pltpu

_N = 4096
_M = 8
_TOPK = 10
_NCAND = 16
_ROWS = 32
_COLS = 128
_NEG = -1.0
_BIGI = 1 << 30


def _tc_body(x_ref, mt_ref, prev_ref,
             change_ref, explained_ref, residual_ref, trip_ref, niche_ref):
    x = x_ref[...]
    prev = prev_ref[...]
    change = x - prev
    change_ref[...] = change

    niche_list = []
    explained = jnp.zeros((_ROWS, _COLS), jnp.float32)
    for j in range(_M):
        nj = jnp.sum(mt_ref[j] * change)
        niche_list.append(nj)
    for j in range(_M):
        explained = explained + niche_list[j] * mt_ref[j]
    explained_ref[...] = explained
    residual_ref[...] = change - explained
    niche_ref[0, :] = jnp.stack(niche_list)

    a = jnp.abs(change)
    fidx = (lax.broadcasted_iota(jnp.int32, (_ROWS, _COLS), 0) * _COLS
            + lax.broadcasted_iota(jnp.int32, (_ROWS, _COLS), 1))
    vals = []
    idxs = []
    for _ in range(_NCAND):
        m = jnp.max(a)
        i = jnp.min(jnp.where(a == m, fidx, _BIGI))
        vals.append(m)
        idxs.append(i)
        a = jnp.where(fidx == i, _NEG, a)

    v16 = jnp.stack(vals)
    i16 = jnp.stack(idxs)
    prod = v16[:, None] * v16[None, :]
    keys = i16[:, None] * _N + i16[None, :]
    rr = lax.broadcasted_iota(jnp.int32, (_NCAND, _NCAND), 0)
    cc = lax.broadcasted_iota(jnp.int32, (_NCAND, _NCAND), 1)
    prod = jnp.where(rr == cc, _NEG, prod)

    tvals = []
    tkeys = []
    for _ in range(_TOPK):
        m = jnp.max(prod)
        k = jnp.min(jnp.where(prod == m, keys, _BIGI))
        tvals.append(m)
        tkeys.append(k)
        prod = jnp.where(keys == k, _NEG, prod)

    max_val = jnp.maximum(tvals[0], 1e-8)
    out = []
    for t in range(_TOPK):
        out.append((tkeys[t] >> 12).astype(jnp.float32))
        out.append((tkeys[t] & (_N - 1)).astype(jnp.float32))
        out.append(tvals[t] / max_val)
    trip_ref[0, :] = jnp.stack(out)


def _run_tc(x2, mt3, prev2):
    return pl.pallas_call(
        _tc_body,
        out_shape=(
            jax.ShapeDtypeStruct((_ROWS, _COLS), jnp.float32),
            jax.ShapeDtypeStruct((_ROWS, _COLS), jnp.float32),
            jax.ShapeDtypeStruct((_ROWS, _COLS), jnp.float32),
            jax.ShapeDtypeStruct((1, 3 * _TOPK), jnp.float32),
            jax.ShapeDtypeStruct((1, _M), jnp.float32),
        ),
    )(x2, mt3, prev2)


@jax.jit
def kernel(current_neuron_state, mechanism_state, prev_state):
    x2 = current_neuron_state.reshape(_ROWS, _COLS)
    prev2 = prev_state.reshape(_ROWS, _COLS)
    mt3 = mechanism_state.T.reshape(_M, _ROWS, _COLS)
    change, explained, residual, trip, niche = _run_tc(x2, mt3, prev2)
    return jnp.concatenate([
        change.reshape(-1), explained.reshape(-1), residual.reshape(-1),
        trip.reshape(-1), niche.reshape(-1),
    ])
